# baseline (speedup 1.0000x reference)
import functools

import jax
import jax.numpy as jnp
from jax import lax
from jax.experimental import pallas as pl
from jax.experimental.pallas import tpu as pltpu
from jax.experimental.pallas import tpu_sc as plsc

B = 16384
F = 26
D = 64
V = 1000000
L = 16
NC = 2
NS = 16
NW = NC * NS
BPW = B // NW
C = 32
CF = C * F
NCHUNK = BPW // C

TBLK = 4096


def _transpose_body(x_ref, o_ref):
    o_ref[...] = x_ref[...].T


def _transpose_table(table_t):
    grid = (pl.cdiv(V, TBLK),)
    return pl.pallas_call(
        _transpose_body,
        grid=grid,
        in_specs=[pl.BlockSpec((D, TBLK), lambda i: (0, i))],
        out_specs=pl.BlockSpec((TBLK, D), lambda i: (i, 0)),
        out_shape=jax.ShapeDtypeStruct((V, D), jnp.float32),
        compiler_params=pltpu.CompilerParams(
            dimension_semantics=("arbitrary",),
        ),
    )(table_t)


def _sc_body(idx_hbm, table_hbm, out_hbm, idx_v, rows_v, out_v, sem):
    wid = lax.axis_index("s") * NC + lax.axis_index("c")
    row0 = wid * BPW

    def chunk(t, carry):
        ibase = (row0 + t * C) * F
        pltpu.sync_copy(idx_hbm.at[pl.ds(ibase, CF)], idx_v)
        pltpu.async_copy(table_hbm.at[idx_v], rows_v, sem).wait()

        def row(b, carry2):
            base = b * F
            for d in range(D // L):
                acc = rows_v[base, pl.ds(d * L, L)]
                for f in range(1, F):
                    acc = acc + rows_v[base + f, pl.ds(d * L, L)]
                out_v[b, pl.ds(d * L, L)] = acc
            return carry2

        lax.fori_loop(0, C, row, 0, unroll=False)
        pltpu.sync_copy(out_v, out_hbm.at[pl.ds(row0 + t * C, C)])
        return carry

    lax.fori_loop(0, NCHUNK, chunk, 0, unroll=False)


@jax.jit
def _encoder_call(idx_flat, table_t):
    table_rm = _transpose_table(table_t)
    mesh = plsc.VectorSubcoreMesh(core_axis_name="c", subcore_axis_name="s")
    run = pl.kernel(
        _sc_body,
        out_type=jax.ShapeDtypeStruct((B, D), jnp.float32),
        mesh=mesh,
        scratch_types=[
            pltpu.VMEM((CF,), jnp.int32),
            pltpu.VMEM((CF, D), jnp.float32),
            pltpu.VMEM((C, D), jnp.float32),
            pltpu.SemaphoreType.DMA,
        ],
        compiler_params=pltpu.CompilerParams(use_tc_tiling_on_sc=False),
    )
    return run(idx_flat, table_rm)


def kernel(indices, table):
    idx_flat = indices.reshape(-1).astype(jnp.int32)
    return _encoder_call(idx_flat, table.T)


# speedup vs baseline: 1.0771x; 1.0771x over previous
import functools

import jax
import jax.numpy as jnp
from jax import lax
from jax.experimental import pallas as pl
from jax.experimental.pallas import tpu as pltpu
from jax.experimental.pallas import tpu_sc as plsc

B = 16384
F = 26
D = 64
V = 1000000
L = 16
NC = 2
NS = 16
NW = NC * NS
BPW = B // NW
C = 32
CF = C * F
NCHUNK = BPW // C

TBLK = 8192


def _transpose_body(x_ref, o_ref):
    x = x_ref[...]
    ii = lax.broadcasted_iota(jnp.int32, (D, D), 0)
    jj = lax.broadcasted_iota(jnp.int32, (D, D), 1)
    eye = jnp.where(ii == jj, 1.0, 0.0).astype(jnp.float32)
    o_ref[...] = lax.dot_general(
        x, eye, (((0,), (0,)), ((), ())),
        preferred_element_type=jnp.float32,
        precision=jax.lax.Precision.DEFAULT,
    )


def _transpose_table(table_t):
    grid = (pl.cdiv(V, TBLK),)
    return pl.pallas_call(
        _transpose_body,
        grid=grid,
        in_specs=[pl.BlockSpec((D, TBLK), lambda i: (0, i))],
        out_specs=pl.BlockSpec((TBLK, D), lambda i: (i, 0)),
        out_shape=jax.ShapeDtypeStruct((V, D), jnp.float32),
        compiler_params=pltpu.CompilerParams(
            dimension_semantics=("arbitrary",),
        ),
    )(table_t)


def _sc_body(idx_hbm, table_hbm, out_hbm, idx_v, rows_v, out_v, sem):
    wid = lax.axis_index("s") * NC + lax.axis_index("c")
    row0 = wid * BPW

    def chunk(t, carry):
        ibase = (row0 + t * C) * F
        pltpu.sync_copy(idx_hbm.at[pl.ds(ibase, CF)], idx_v)
        pltpu.async_copy(table_hbm.at[idx_v], rows_v, sem).wait()

        def row(b, carry2):
            base = b * F
            for d in range(D // L):
                acc = rows_v[base, pl.ds(d * L, L)]
                for f in range(1, F):
                    acc = acc + rows_v[base + f, pl.ds(d * L, L)]
                out_v[b, pl.ds(d * L, L)] = acc
            return carry2

        lax.fori_loop(0, C, row, 0, unroll=False)
        pltpu.sync_copy(out_v, out_hbm.at[pl.ds(row0 + t * C, C)])
        return carry

    lax.fori_loop(0, NCHUNK, chunk, 0, unroll=False)


@jax.jit
def _encoder_call(idx_flat, table_t):
    table_rm = _transpose_table(table_t)
    mesh = plsc.VectorSubcoreMesh(core_axis_name="c", subcore_axis_name="s")
    run = pl.kernel(
        _sc_body,
        out_type=jax.ShapeDtypeStruct((B, D), jnp.float32),
        mesh=mesh,
        scratch_types=[
            pltpu.VMEM((CF,), jnp.int32),
            pltpu.VMEM((CF, D), jnp.float32),
            pltpu.VMEM((C, D), jnp.float32),
            pltpu.SemaphoreType.DMA,
        ],
        compiler_params=pltpu.CompilerParams(use_tc_tiling_on_sc=False),
    )
    return run(idx_flat, table_rm)


def kernel(indices, table):
    idx_flat = indices.reshape(-1).astype(jnp.int32)
    return _encoder_call(idx_flat, table.T)


# speedup vs baseline: 1.1961x; 1.1105x over previous
import functools

import jax
import jax.numpy as jnp
from jax import lax
from jax.experimental import pallas as pl
from jax.experimental.pallas import tpu as pltpu
from jax.experimental.pallas import tpu_sc as plsc

B = 16384
F = 26
D = 64
V = 1000000
L = 16
NC = 2
NS = 16
NW = NC * NS
BPW = B // NW
C = 32
CF = C * F
NCHUNK = BPW // C

TBLK = 32768


def _transpose_body(x_ref, o_ref):
    x = x_ref[...]
    ii = lax.broadcasted_iota(jnp.int32, (D, D), 0)
    jj = lax.broadcasted_iota(jnp.int32, (D, D), 1)
    eye = jnp.where(ii == jj, 1.0, 0.0).astype(jnp.float32)
    o_ref[...] = lax.dot_general(
        x, eye, (((0,), (0,)), ((), ())),
        preferred_element_type=jnp.float32,
        precision=jax.lax.Precision.DEFAULT,
    )


def _transpose_table(table_t):
    grid = (pl.cdiv(V, TBLK),)
    return pl.pallas_call(
        _transpose_body,
        grid=grid,
        in_specs=[pl.BlockSpec((D, TBLK), lambda i: (0, i))],
        out_specs=pl.BlockSpec((TBLK, D), lambda i: (i, 0)),
        out_shape=jax.ShapeDtypeStruct((V, D), jnp.float32),
        compiler_params=pltpu.CompilerParams(
            dimension_semantics=("arbitrary",),
            vmem_limit_bytes=100 * 1024 * 1024,
        ),
    )(table_t)


def _sc_body(idx_hbm, table_hbm, out_hbm, idx_all, rows2, out2,
             sem_g0, sem_g1, sem_o0, sem_o1):
    wid = lax.axis_index("s") * NC + lax.axis_index("c")
    row0 = wid * BPW
    sem_g = (sem_g0, sem_g1)
    sem_o = (sem_o0, sem_o1)

    pltpu.sync_copy(idx_hbm.at[pl.ds(row0 * F, BPW * F)], idx_all)

    def gather_start(g, b):
        pltpu.async_copy(
            table_hbm.at[idx_all.at[pl.ds(g * CF, CF)]], rows2.at[b], sem_g[b])

    def gather_wait(b):
        pltpu.make_async_copy(
            table_hbm.at[idx_all.at[pl.ds(0, CF)]], rows2.at[b], sem_g[b]).wait()

    def out_start(g, b):
        pltpu.async_copy(out2.at[b], out_hbm.at[pl.ds(row0 + g * C, C)], sem_o[b])

    def out_wait(b):
        pltpu.make_async_copy(
            out2.at[b], out_hbm.at[pl.ds(row0, C)], sem_o[b]).wait()

    gather_start(0, 0)

    def pair(i, carry):
        for b in (0, 1):
            g = 2 * i + b

            @pl.when(g + 1 < NCHUNK)
            def _():
                gather_start(g + 1, 1 - b)

            gather_wait(b)

            @pl.when(g >= 2)
            def _():
                out_wait(b)

            rows_v = rows2.at[b]
            out_v = out2.at[b]

            def row(r, carry2):
                base = r * F
                for d in range(D // L):
                    acc = rows_v[base, pl.ds(d * L, L)]
                    for f in range(1, F):
                        acc = acc + rows_v[base + f, pl.ds(d * L, L)]
                    out_v[r, pl.ds(d * L, L)] = acc
                return carry2

            lax.fori_loop(0, C, row, 0, unroll=False)
            out_start(g, b)
        return carry

    lax.fori_loop(0, NCHUNK // 2, pair, 0, unroll=False)
    out_wait(0)
    out_wait(1)


@jax.jit
def _encoder_call(idx_flat, table_t):
    table_rm = _transpose_table(table_t)
    mesh = plsc.VectorSubcoreMesh(core_axis_name="c", subcore_axis_name="s")
    run = pl.kernel(
        _sc_body,
        out_type=jax.ShapeDtypeStruct((B, D), jnp.float32),
        mesh=mesh,
        scratch_types=[
            pltpu.VMEM((BPW * F,), jnp.int32),
            pltpu.VMEM((2, CF, D), jnp.float32),
            pltpu.VMEM((2, C, D), jnp.float32),
            pltpu.SemaphoreType.DMA,
            pltpu.SemaphoreType.DMA,
            pltpu.SemaphoreType.DMA,
            pltpu.SemaphoreType.DMA,
        ],
        compiler_params=pltpu.CompilerParams(use_tc_tiling_on_sc=False),
    )
    return run(idx_flat, table_rm)


def kernel(indices, table):
    idx_flat = indices.reshape(-1).astype(jnp.int32)
    return _encoder_call(idx_flat, table.T)


# speedup vs baseline: 2.3043x; 1.9265x over previous
import functools

import jax
import jax.numpy as jnp
from jax import lax
from jax.experimental import pallas as pl
from jax.experimental.pallas import tpu as pltpu
from jax.experimental.pallas import tpu_sc as plsc

B = 16384
F = 26
D = 64
V = 1000000
VP = 1048576
L = 16
NC = 2
NS = 16
NW = NC * NS
BPW = B // NW
C = 16
CF = C * F
NCHUNK = BPW // C

TQ = 16384


def _transpose_body(x_ref, o_ref):
    ii = lax.broadcasted_iota(jnp.int32, (D, D), 0)
    jj = lax.broadcasted_iota(jnp.int32, (D, D), 1)
    eye = jnp.where(ii == jj, 1.0, 0.0).astype(jnp.float32)
    o_ref[:, 0:D] = lax.dot_general(
        x_ref[...], eye, (((0,), (0,)), ((), ())),
        preferred_element_type=jnp.float32,
        precision=jax.lax.Precision.DEFAULT,
    )


def _transpose_table(table_t):
    grid = (pl.cdiv(V, TQ),)
    return pl.pallas_call(
        _transpose_body,
        grid=grid,
        in_specs=[pl.BlockSpec((D, TQ), lambda i: (0, i))],
        out_specs=pl.BlockSpec((TQ, 2 * D), lambda i: (i, 0)),
        out_shape=jax.ShapeDtypeStruct((VP, 2 * D), jnp.float32),
        compiler_params=pltpu.CompilerParams(
            dimension_semantics=("arbitrary",),
            vmem_limit_bytes=100 * 1024 * 1024,
        ),
    )(table_t)


def _sc_body(idx_hbm, table_hbm, out_hbm, idx_all, rows2, out2,
             sem_g0, sem_g1, sem_o0, sem_o1):
    wid = lax.axis_index("s") * NC + lax.axis_index("c")
    row0 = wid * BPW
    sem_g = (sem_g0, sem_g1)
    sem_o = (sem_o0, sem_o1)

    pltpu.sync_copy(idx_hbm.at[pl.ds(row0 * F, BPW * F)], idx_all)

    def gather_start(g, b):
        pltpu.async_copy(
            table_hbm.at[idx_all.at[pl.ds(g * CF, CF)]], rows2.at[b], sem_g[b])

    def gather_wait(b):
        pltpu.make_async_copy(
            table_hbm.at[idx_all.at[pl.ds(0, CF)]], rows2.at[b], sem_g[b]).wait()

    def out_start(g, b):
        pltpu.async_copy(out2.at[b], out_hbm.at[pl.ds(row0 + g * C, C)], sem_o[b])

    def out_wait(b):
        pltpu.make_async_copy(
            out2.at[b], out_hbm.at[pl.ds(row0, C)], sem_o[b]).wait()

    gather_start(0, 0)

    def pair(i, carry):
        for b in (0, 1):
            g = 2 * i + b

            @pl.when(g + 1 < NCHUNK)
            def _():
                gather_start(g + 1, 1 - b)

            gather_wait(b)

            @pl.when(g >= 2)
            def _():
                out_wait(b)

            rows_v = rows2.at[b]
            out_v = out2.at[b]

            def row(r, carry2):
                base = r * F
                for d in range(D // L):
                    acc = rows_v[base, pl.ds(d * L, L)]
                    for f in range(1, F):
                        acc = acc + rows_v[base + f, pl.ds(d * L, L)]
                    out_v[r, pl.ds(d * L, L)] = acc
                return carry2

            lax.fori_loop(0, C, row, 0, unroll=False)
            out_start(g, b)
        return carry

    lax.fori_loop(0, NCHUNK // 2, pair, 0, unroll=False)
    out_wait(0)
    out_wait(1)


@jax.jit
def _encoder_call(idx_flat, table_t):
    table_sc = _transpose_table(table_t)
    mesh = plsc.VectorSubcoreMesh(core_axis_name="c", subcore_axis_name="s")
    run = pl.kernel(
        _sc_body,
        out_type=jax.ShapeDtypeStruct((B, D), jnp.float32),
        mesh=mesh,
        scratch_types=[
            pltpu.VMEM((BPW * F,), jnp.int32),
            pltpu.VMEM((2, CF, 2 * D), jnp.float32),
            pltpu.VMEM((2, C, D), jnp.float32),
            pltpu.SemaphoreType.DMA,
            pltpu.SemaphoreType.DMA,
            pltpu.SemaphoreType.DMA,
            pltpu.SemaphoreType.DMA,
        ],
        compiler_params=pltpu.CompilerParams(use_tc_tiling_on_sc=False),
    )
    return run(idx_flat, table_sc)


def kernel(indices, table):
    idx_flat = indices.reshape(-1).astype(jnp.int32)
    return _encoder_call(idx_flat, table.T)


# speedup vs baseline: 2.4086x; 1.0453x over previous
import functools

import jax
import jax.numpy as jnp
from jax import lax
from jax.experimental import pallas as pl
from jax.experimental.pallas import tpu as pltpu
from jax.experimental.pallas import tpu_sc as plsc

B = 16384
F = 26
D = 64
V = 1000000
VP = 1048576
L = 16
NC = 2
NS = 16
NW = NC * NS
BPW = B // NW
C = 32
CF = C * F
NCHUNK = BPW // C

TQ = 16384


def _transpose_body(x_ref, o_ref):
    ii = lax.broadcasted_iota(jnp.int32, (D, D), 0)
    jj = lax.broadcasted_iota(jnp.int32, (D, D), 1)
    eye = jnp.where(ii == jj, 1.0, 0.0).astype(jnp.float32)
    o_ref[:, 0:D] = lax.dot_general(
        x_ref[...], eye, (((0,), (0,)), ((), ())),
        preferred_element_type=jnp.float32,
        precision=jax.lax.Precision.DEFAULT,
    )


def _transpose_table(table_t):
    grid = (pl.cdiv(V, TQ),)
    return pl.pallas_call(
        _transpose_body,
        grid=grid,
        in_specs=[pl.BlockSpec((D, TQ), lambda i: (0, i))],
        out_specs=pl.BlockSpec((TQ, 2 * D), lambda i: (i, 0)),
        out_shape=jax.ShapeDtypeStruct((VP, 2 * D), jnp.float32),
        compiler_params=pltpu.CompilerParams(
            dimension_semantics=("arbitrary",),
            vmem_limit_bytes=100 * 1024 * 1024,
        ),
    )(table_t)


def _sc_body(idx_hbm, table_hbm, out_hbm, idx_all, rows2, out2,
             sem_g0, sem_g1, sem_o0, sem_o1):
    wid = lax.axis_index("s") * NC + lax.axis_index("c")
    row0 = wid * BPW
    sem_g = (sem_g0, sem_g1)
    sem_o = (sem_o0, sem_o1)

    pltpu.sync_copy(idx_hbm.at[pl.ds(row0 * F, BPW * F)], idx_all)

    def gather_start(g, b):
        pltpu.async_copy(
            table_hbm.at[idx_all.at[pl.ds(g * CF, CF)]], rows2.at[b], sem_g[b])

    def gather_wait(b):
        pltpu.make_async_copy(
            table_hbm.at[idx_all.at[pl.ds(0, CF)]], rows2.at[b], sem_g[b]).wait()

    def out_start(g, b):
        pltpu.async_copy(out2.at[b], out_hbm.at[pl.ds(row0 + g * C, C)], sem_o[b])

    def out_wait(b):
        pltpu.make_async_copy(
            out2.at[b], out_hbm.at[pl.ds(row0, C)], sem_o[b]).wait()

    gather_start(0, 0)

    def pair(i, carry):
        for b in (0, 1):
            g = 2 * i + b

            @pl.when(g + 1 < NCHUNK)
            def _():
                gather_start(g + 1, 1 - b)

            gather_wait(b)

            @pl.when(g >= 2)
            def _():
                out_wait(b)

            rows_v = rows2.at[b]
            out_v = out2.at[b]

            def row(r, carry2):
                base = r * F
                for d in range(D // L):
                    acc = rows_v[base, pl.ds(d * L, L)]
                    for f in range(1, F):
                        acc = acc + rows_v[base + f, pl.ds(d * L, L)]
                    out_v[r, pl.ds(d * L, L)] = acc
                return carry2

            lax.fori_loop(0, C, row, 0, unroll=False)
            out_start(g, b)
        return carry

    lax.fori_loop(0, NCHUNK // 2, pair, 0, unroll=False)
    out_wait(0)
    out_wait(1)


@jax.jit
def _encoder_call(idx_flat, table_t):
    table_sc = _transpose_table(table_t).reshape(2 * VP, D)
    mesh = plsc.VectorSubcoreMesh(core_axis_name="c", subcore_axis_name="s")
    run = pl.kernel(
        _sc_body,
        out_type=jax.ShapeDtypeStruct((B, D), jnp.float32),
        mesh=mesh,
        scratch_types=[
            pltpu.VMEM((BPW * F,), jnp.int32),
            pltpu.VMEM((2, CF, D), jnp.float32),
            pltpu.VMEM((2, C, D), jnp.float32),
            pltpu.SemaphoreType.DMA,
            pltpu.SemaphoreType.DMA,
            pltpu.SemaphoreType.DMA,
            pltpu.SemaphoreType.DMA,
        ],
        compiler_params=pltpu.CompilerParams(use_tc_tiling_on_sc=False),
    )
    return run(idx_flat, table_sc)


def kernel(indices, table):
    idx_flat = indices.reshape(-1).astype(jnp.int32) * 2
    return _encoder_call(idx_flat, table.T)


# speedup vs baseline: 2.4501x; 1.0172x over previous
import functools

import jax
import jax.numpy as jnp
from jax import lax
from jax.experimental import pallas as pl
from jax.experimental.pallas import tpu as pltpu
from jax.experimental.pallas import tpu_sc as plsc

B = 16384
F = 26
D = 64
V = 1000000
VP = 1048576
L = 16
NC = 2
NS = 16
NW = NC * NS
BPW = B // NW
C = 32
CF = C * F
NCHUNK = BPW // C

TQ = 32768


def _transpose_body(x_ref, o_ref):
    ii = lax.broadcasted_iota(jnp.int32, (D, D), 0)
    jj = lax.broadcasted_iota(jnp.int32, (D, D), 1)
    eye = jnp.where(ii == jj, 1.0, 0.0).astype(jnp.float32)
    o_ref[:, 0:D] = lax.dot_general(
        x_ref[...], eye, (((0,), (0,)), ((), ())),
        preferred_element_type=jnp.float32,
        precision=jax.lax.Precision.DEFAULT,
    )


def _transpose_table(table_t):
    grid = (pl.cdiv(V, TQ),)
    return pl.pallas_call(
        _transpose_body,
        grid=grid,
        in_specs=[pl.BlockSpec((D, TQ), lambda i: (0, i))],
        out_specs=pl.BlockSpec((TQ, 2 * D), lambda i: (i, 0)),
        out_shape=jax.ShapeDtypeStruct((VP, 2 * D), jnp.float32),
        compiler_params=pltpu.CompilerParams(
            dimension_semantics=("arbitrary",),
            vmem_limit_bytes=100 * 1024 * 1024,
        ),
    )(table_t)


def _sc_body(idx_hbm, table_hbm, out_hbm, idx_all, rows2, out2,
             sem_g0, sem_g1, sem_o0, sem_o1):
    wid = lax.axis_index("s") * NC + lax.axis_index("c")
    row0 = wid * BPW
    sem_g = (sem_g0, sem_g1)
    sem_o = (sem_o0, sem_o1)

    pltpu.sync_copy(idx_hbm.at[pl.ds(row0 * F, BPW * F)], idx_all)

    def gather_start(g, b):
        pltpu.async_copy(
            table_hbm.at[idx_all.at[pl.ds(g * CF, CF)]], rows2.at[b], sem_g[b])

    def gather_wait(b):
        pltpu.make_async_copy(
            table_hbm.at[idx_all.at[pl.ds(0, CF)]], rows2.at[b], sem_g[b]).wait()

    def out_start(g, b):
        pltpu.async_copy(out2.at[b], out_hbm.at[pl.ds(row0 + g * C, C)], sem_o[b])

    def out_wait(b):
        pltpu.make_async_copy(
            out2.at[b], out_hbm.at[pl.ds(row0, C)], sem_o[b]).wait()

    gather_start(0, 0)

    def pair(i, carry):
        for b in (0, 1):
            g = 2 * i + b

            @pl.when(g + 1 < NCHUNK)
            def _():
                gather_start(g + 1, 1 - b)

            gather_wait(b)

            @pl.when(g >= 2)
            def _():
                out_wait(b)

            rows_v = rows2.at[b]
            out_v = out2.at[b]

            def row(r, carry2):
                base = r * F
                for d in range(D // L):
                    acc = rows_v[base, pl.ds(d * L, L)]
                    for f in range(1, F):
                        acc = acc + rows_v[base + f, pl.ds(d * L, L)]
                    out_v[r, pl.ds(d * L, L)] = acc
                return carry2

            lax.fori_loop(0, C, row, 0, unroll=False)
            out_start(g, b)
        return carry

    lax.fori_loop(0, NCHUNK // 2, pair, 0, unroll=False)
    out_wait(0)
    out_wait(1)


@jax.jit
def _encoder_call(idx_flat, table_t):
    table_sc = _transpose_table(table_t).reshape(2 * VP, D)
    mesh = plsc.VectorSubcoreMesh(core_axis_name="c", subcore_axis_name="s")
    run = pl.kernel(
        _sc_body,
        out_type=jax.ShapeDtypeStruct((B, D), jnp.float32),
        mesh=mesh,
        scratch_types=[
            pltpu.VMEM((BPW * F,), jnp.int32),
            pltpu.VMEM((2, CF, D), jnp.float32),
            pltpu.VMEM((2, C, D), jnp.float32),
            pltpu.SemaphoreType.DMA,
            pltpu.SemaphoreType.DMA,
            pltpu.SemaphoreType.DMA,
            pltpu.SemaphoreType.DMA,
        ],
        compiler_params=pltpu.CompilerParams(use_tc_tiling_on_sc=False),
    )
    return run(idx_flat, table_sc)


def kernel(indices, table):
    idx_flat = indices.reshape(-1).astype(jnp.int32) * 2
    return _encoder_call(idx_flat, table.T)
